# baseline (speedup 1.0000x reference)
import functools

import jax
import jax.numpy as jnp
from jax import lax
from jax.experimental import pallas as pl
from jax.experimental.pallas import tpu as pltpu
from jax.experimental.pallas import tpu_sc as plsc

DIM = 64
LIST_LEN = 200
NUM_LISTS = 7
LANES = 16
IDS_PER_WORKER = 50
NUM_GATHER_WORKERS = 28
NBUF = 12


def _sc_gather(id_lists, tt):
    mesh = plsc.VectorSubcoreMesh(
        core_axis_name="c", subcore_axis_name="s",
        num_cores=2, num_subcores=16)

    @functools.partial(
        pl.kernel,
        mesh=mesh,
        out_type=jax.ShapeDtypeStruct((NUM_GATHER_WORKERS, 1, DIM),
                                      jnp.float32),
        compiler_params=pltpu.CompilerParams(needs_layout_passes=False),
        scratch_types=[
            pltpu.VMEM((LIST_LEN,), jnp.int32),
            *[pltpu.VMEM((DIM, 128), jnp.float32) for _ in range(NBUF)],
            pltpu.VMEM((1, DIM), jnp.float32),
            *[pltpu.SemaphoreType.DMA for _ in range(NBUF)],
        ],
    )
    def enc(l0, l1, l2, l3, l4, l5, l6, tt_hbm, out_hbm, idx_v, *rest):
        bufs = list(rest[:NBUF])
        outv = rest[NBUF]
        sems = list(rest[NBUF + 1:])
        wid = lax.axis_index("s") * 2 + lax.axis_index("c")

        @pl.when(wid < NUM_GATHER_WORKERS)
        def _lists():
            lst = wid // 4
            q = wid % 4
            for i, ref in enumerate((l0, l1, l2, l3, l4, l5, l6)):
                @pl.when(lst == i)
                def _(ref=ref):
                    pltpu.sync_copy(ref, idx_v)

            iota = lax.iota(jnp.int32, LANES)
            base = q * IDS_PER_WORKER
            ivs = []
            for g in range(4):
                gidx = iota + (base + LANES * g)
                if g == 3:
                    gidx = jnp.minimum(gidx, LIST_LEN - 1)
                ivs.append(plsc.load_gather(idx_v, [gidx]))

            cps = [None] * IDS_PER_WORKER
            lanes = [None] * IDS_PER_WORKER

            def fire(k, slot):
                r = ivs[k // LANES][k % LANES]
                jt = pl.multiple_of((r >> 7) << 7, 128)
                lanes[k] = r & 127
                cps[k] = pltpu.async_copy(
                    tt_hbm.at[:, pl.ds(jt, 128)], bufs[slot], sems[slot])

            for k in range(NBUF):
                fire(k, k)
            z = jnp.zeros((LANES,), jnp.float32)
            accs = [z, z, z, z]
            iotas = [iota + LANES * c for c in range(4)]
            for k in range(IDS_PER_WORKER):
                slot = k % NBUF
                cps[k].wait()
                colidx = jnp.full((LANES,), lanes[k], jnp.int32)
                for c in range(4):
                    rows = plsc.load_gather(bufs[slot], [iotas[c], colidx])
                    accs[c] = accs[c] + rows
                if k + NBUF < IDS_PER_WORKER:
                    fire(k + NBUF, slot)
            for c in range(4):
                outv[0, pl.ds(LANES * c, LANES)] = accs[c]
            pltpu.sync_copy(outv, out_hbm.at[NUM_LISTS * q + lst])

    return enc(*id_lists, tt)


def _tc_combine_body(svec_ref, x_ref, w1p_ref, w2t_ref, b2_ref, o_ref):
    scale = jnp.float32(1.0 / LIST_LEN)
    sums = (x_ref[pl.ds(0, NUM_LISTS), 0, :]
            + x_ref[pl.ds(NUM_LISTS, NUM_LISTS), 0, :]
            + x_ref[pl.ds(2 * NUM_LISTS, NUM_LISTS), 0, :]
            + x_ref[pl.ds(3 * NUM_LISTS, NUM_LISTS), 0, :]) * scale
    o_ref[pl.ds(0, NUM_LISTS), :] = sums
    s = jnp.stack([svec_ref[k] for k in range(8)])[None, :]
    h = jnp.maximum(
        jnp.dot(s, w1p_ref[...], precision=lax.Precision.HIGHEST), 0.0)
    out = jnp.dot(h, w2t_ref[...], precision=lax.Precision.HIGHEST)
    o_ref[pl.ds(NUM_LISTS, 1), :] = out + b2_ref[...][None, :]


def _tc_combine(svec, partials, w1p, w2t, b2):
    return pl.pallas_call(
        _tc_combine_body,
        in_specs=[
            pl.BlockSpec(memory_space=pltpu.SMEM),
            pl.BlockSpec(memory_space=pltpu.VMEM),
            pl.BlockSpec(memory_space=pltpu.VMEM),
            pl.BlockSpec(memory_space=pltpu.VMEM),
            pl.BlockSpec(memory_space=pltpu.VMEM),
        ],
        out_shape=jax.ShapeDtypeStruct((NUM_LISTS + 1, DIM), jnp.float32),
    )(svec, partials, w1p, w2t, b2)


def kernel(self_main_ids, self_field_ids, self_graveyard_ids, self_banish_ids,
           opp_fields_ids, opp_graveyard_ids, opp_banish_ids,
           self_banish_verso, opp_banish_verso, opp_fields_verso_card,
           phase_id, lp, adv_lp, opp_main,
           table, W1, b1, W2, b2):
    id_lists = [self_main_ids.astype(jnp.int32),
                self_field_ids.astype(jnp.int32),
                opp_fields_ids.astype(jnp.int32),
                self_graveyard_ids.astype(jnp.int32),
                self_banish_ids.astype(jnp.int32),
                opp_graveyard_ids.astype(jnp.int32),
                opp_banish_ids.astype(jnp.int32)]
    svec = jnp.stack([phase_id, opp_main, lp, adv_lp,
                      opp_fields_verso_card, self_banish_verso,
                      opp_banish_verso, 1]).astype(jnp.float32)
    w1p = jnp.concatenate([W1.T, b1[None, :]], axis=0)
    w2t = W2.T

    partials = _sc_gather(id_lists, table.T)
    out = _tc_combine(svec, partials, w1p, w2t, b2)
    return out.reshape(NUM_LISTS * DIM + DIM)


# speedup vs baseline: 1.0076x; 1.0076x over previous
import functools

import jax
import jax.numpy as jnp
from jax import lax
from jax.experimental import pallas as pl
from jax.experimental.pallas import tpu as pltpu
from jax.experimental.pallas import tpu_sc as plsc

DIM = 64
LIST_LEN = 200
NUM_LISTS = 7
LANES = 16
IDS_PER_WORKER = 50
NUM_GATHER_WORKERS = 28
NBUF = 10


def _sc_gather(id_lists, tt):
    mesh = plsc.VectorSubcoreMesh(
        core_axis_name="c", subcore_axis_name="s",
        num_cores=2, num_subcores=16)

    @functools.partial(
        pl.kernel,
        mesh=mesh,
        out_type=jax.ShapeDtypeStruct((NUM_GATHER_WORKERS, 1, DIM),
                                      jnp.float32),
        compiler_params=pltpu.CompilerParams(needs_layout_passes=False),
        scratch_types=[
            pltpu.VMEM((LIST_LEN,), jnp.int32),
            *[pltpu.VMEM((DIM, 128), jnp.float32) for _ in range(NBUF)],
            pltpu.VMEM((1, DIM), jnp.float32),
            *[pltpu.SemaphoreType.DMA for _ in range(NBUF)],
        ],
    )
    def enc(l0, l1, l2, l3, l4, l5, l6, tt_hbm, out_hbm, idx_v, *rest):
        bufs = list(rest[:NBUF])
        outv = rest[NBUF]
        sems = list(rest[NBUF + 1:])
        wid = lax.axis_index("s") * 2 + lax.axis_index("c")

        @pl.when(wid < NUM_GATHER_WORKERS)
        def _lists():
            lst = wid // 4
            q = wid % 4
            for i, ref in enumerate((l0, l1, l2, l3, l4, l5, l6)):
                @pl.when(lst == i)
                def _(ref=ref):
                    pltpu.sync_copy(ref, idx_v)

            iota = lax.iota(jnp.int32, LANES)
            base = q * IDS_PER_WORKER
            ivs = []
            for g in range(4):
                gidx = iota + (base + LANES * g)
                if g == 3:
                    gidx = jnp.minimum(gidx, LIST_LEN - 1)
                ivs.append(plsc.load_gather(idx_v, [gidx]))

            cps = [None] * IDS_PER_WORKER
            lanes = [None] * IDS_PER_WORKER

            def fire(k, slot):
                r = ivs[k // LANES][k % LANES]
                jt = pl.multiple_of((r >> 7) << 7, 128)
                lanes[k] = r & 127
                cps[k] = pltpu.async_copy(
                    tt_hbm.at[:, pl.ds(jt, 128)], bufs[slot], sems[slot])

            for k in range(NBUF):
                fire(k, k)
            z = jnp.zeros((LANES,), jnp.float32)
            accs = [z, z, z, z]
            iotas = [iota + LANES * c for c in range(4)]
            for k in range(IDS_PER_WORKER):
                slot = k % NBUF
                cps[k].wait()
                colidx = jnp.full((LANES,), lanes[k], jnp.int32)
                for c in range(4):
                    rows = plsc.load_gather(bufs[slot], [iotas[c], colidx])
                    accs[c] = accs[c] + rows
                if k + NBUF < IDS_PER_WORKER:
                    fire(k + NBUF, slot)
            for c in range(4):
                outv[0, pl.ds(LANES * c, LANES)] = accs[c]
            pltpu.sync_copy(outv, out_hbm.at[NUM_LISTS * q + lst])

    return enc(*id_lists, tt)


def _tc_combine_body(svec_ref, x_ref, w1p_ref, w2t_ref, b2_ref, o_ref):
    scale = jnp.float32(1.0 / LIST_LEN)
    sums = (x_ref[pl.ds(0, NUM_LISTS), 0, :]
            + x_ref[pl.ds(NUM_LISTS, NUM_LISTS), 0, :]
            + x_ref[pl.ds(2 * NUM_LISTS, NUM_LISTS), 0, :]
            + x_ref[pl.ds(3 * NUM_LISTS, NUM_LISTS), 0, :]) * scale
    o_ref[pl.ds(0, NUM_LISTS), :] = sums
    s = jnp.stack([svec_ref[k] for k in range(8)])[None, :]
    h = jnp.maximum(
        jnp.dot(s, w1p_ref[...], precision=lax.Precision.HIGHEST), 0.0)
    out = jnp.dot(h, w2t_ref[...], precision=lax.Precision.HIGHEST)
    o_ref[pl.ds(NUM_LISTS, 1), :] = out + b2_ref[...][None, :]


def _tc_combine(svec, partials, w1p, w2t, b2):
    return pl.pallas_call(
        _tc_combine_body,
        in_specs=[
            pl.BlockSpec(memory_space=pltpu.SMEM),
            pl.BlockSpec(memory_space=pltpu.VMEM),
            pl.BlockSpec(memory_space=pltpu.VMEM),
            pl.BlockSpec(memory_space=pltpu.VMEM),
            pl.BlockSpec(memory_space=pltpu.VMEM),
        ],
        out_shape=jax.ShapeDtypeStruct((NUM_LISTS + 1, DIM), jnp.float32),
    )(svec, partials, w1p, w2t, b2)


def kernel(self_main_ids, self_field_ids, self_graveyard_ids, self_banish_ids,
           opp_fields_ids, opp_graveyard_ids, opp_banish_ids,
           self_banish_verso, opp_banish_verso, opp_fields_verso_card,
           phase_id, lp, adv_lp, opp_main,
           table, W1, b1, W2, b2):
    id_lists = [self_main_ids.astype(jnp.int32),
                self_field_ids.astype(jnp.int32),
                opp_fields_ids.astype(jnp.int32),
                self_graveyard_ids.astype(jnp.int32),
                self_banish_ids.astype(jnp.int32),
                opp_graveyard_ids.astype(jnp.int32),
                opp_banish_ids.astype(jnp.int32)]
    svec = jnp.stack([phase_id, opp_main, lp, adv_lp,
                      opp_fields_verso_card, self_banish_verso,
                      opp_banish_verso, 1]).astype(jnp.float32)
    w1p = jnp.concatenate([W1.T, b1[None, :]], axis=0)
    w2t = W2.T

    partials = _sc_gather(id_lists, table.T)
    out = _tc_combine(svec, partials, w1p, w2t, b2)
    return out.reshape(NUM_LISTS * DIM + DIM)
